# baseline (speedup 1.0000x reference)
import functools

import jax
import jax.numpy as jnp
from jax import lax
from jax.experimental import pallas as pl
from jax.experimental.pallas import tpu as pltpu
from jax.experimental.pallas import tpu_sc as plsc

_N = 10000
_M = 1024
_NNZ = 160000
_EMB = 128
_NC = 4
_K = 8000
_W = 144
_ZR = _N + 16
_ZTOT = _ZR * _M
_NTILES = 32
_ZCHUNK = _ZTOT // _NTILES
_PPT_SEG = _NNZ // _NTILES
_PPT_SEN = _NNZ // 16
_SENT = jnp.float32(-2.0e30)

def _segsum_body(xa, v, e, out, idx_v, idx_e, rows, idx_v8, idx_e8, rows8,
                 zbuf, acc, sem):
    c = lax.axis_index("c")
    s = lax.axis_index("s")
    wid = c * 16 + s

    def _zero(i, _):
        r = i // 9
        j = i % 9
        zbuf[r, pl.ds(j * 16, 16)] = jnp.zeros((16,), jnp.float32)
        return 0

    lax.fori_loop(0, 64 * 9, _zero, 0)
    pltpu.sync_copy(zbuf, acc.at[pl.ds(s * 64, 64)])
    plsc.subcore_barrier()

    base0 = wid * _PPT_SEG

    def _chunk(j, _):
        b = base0 + j * 128
        pltpu.sync_copy(v.at[pl.ds(b, 128)], idx_v)
        pltpu.sync_copy(e.at[pl.ds(b, 128)], idx_e)
        pltpu.async_copy(xa.at[idx_v], rows, sem).wait()
        pltpu.sync_copy(rows, acc.at[idx_e], add=True)
        return 0

    lax.fori_loop(0, 39, _chunk, 0)
    bt = base0 + 39 * 128
    pltpu.sync_copy(v.at[pl.ds(bt, 8)], idx_v8)
    pltpu.sync_copy(e.at[pl.ds(bt, 8)], idx_e8)
    pltpu.async_copy(xa.at[idx_v8], rows8, sem).wait()
    pltpu.sync_copy(rows8, acc.at[idx_e8], add=True)

    plsc.subcore_barrier()
    pltpu.sync_copy(acc.at[pl.ds(s * 64, 64)], out.at[c, pl.ds(s * 64, 64)])


def _sentinel_body(idxf, zout, zbuf, ibuf, sbuf, cbuf, ibuf16, sbuf16, cbuf16):
    c = lax.axis_index("c")
    s = lax.axis_index("s")
    wid = c * 16 + s

    def _zero(i, _):
        zbuf[pl.ds(i * 16, 16)] = jnp.zeros((16,), jnp.float32)
        return 0

    lax.fori_loop(0, 512, _zero, 0)
    for j in range(8):
        cbuf[pl.ds(j * 16, 16)] = jnp.full((16,), _SENT, jnp.float32)
    cbuf16[pl.ds(0, 16)] = jnp.full((16,), _SENT, jnp.float32)

    zlo = wid * _ZCHUNK

    def _fill(i, _):
        pltpu.sync_copy(zbuf, zout.at[pl.ds(zlo + i * 8192, 8192)])
        return 0

    lax.fori_loop(0, 39, _fill, 0)
    pltpu.sync_copy(zbuf.at[pl.ds(0, 1024)],
                    zout.at[pl.ds(zlo + 39 * 8192, 1024)])
    plsc.subcore_barrier()

    half = _ZTOT // 2
    hlo = c * half
    hhi = hlo + half
    dump = _N * _M + wid
    pbase = s * _PPT_SEN

    def _sel(buf_i, buf_s, nsub):
        for jj in range(nsub):
            vv = buf_i[pl.ds(jj * 16, 16)]
            inr = (vv >= hlo) & (vv < hhi)
            buf_s[pl.ds(jj * 16, 16)] = jnp.where(
                inr, vv, jnp.zeros((16,), jnp.int32) + dump)

    def _scat(j, _):
        b = pbase + j * 128
        pltpu.sync_copy(idxf.at[pl.ds(b, 128)], ibuf)
        _sel(ibuf, sbuf, 8)
        pltpu.sync_copy(cbuf, zout.at[sbuf])
        return 0

    lax.fori_loop(0, 78, _scat, 0)
    bt = pbase + 78 * 128
    pltpu.sync_copy(idxf.at[pl.ds(bt, 16)], ibuf16)
    _sel(ibuf16, sbuf16, 1)
    pltpu.sync_copy(cbuf16, zout.at[sbuf16])


@functools.cache
def _sc_kernels():
    mesh = plsc.VectorSubcoreMesh(core_axis_name="c", subcore_axis_name="s",
                                  num_cores=2, num_subcores=16)
    params = pltpu.CompilerParams(use_tc_tiling_on_sc=False)
    segsum = pl.kernel(
        _segsum_body,
        out_type=jax.ShapeDtypeStruct((2, _M, _W), jnp.float32),
        mesh=mesh,
        compiler_params=params,
        scratch_types=[
            pltpu.VMEM((128,), jnp.int32),
            pltpu.VMEM((128,), jnp.int32),
            pltpu.VMEM((128, _W), jnp.float32),
            pltpu.VMEM((8,), jnp.int32),
            pltpu.VMEM((8,), jnp.int32),
            pltpu.VMEM((8, _W), jnp.float32),
            pltpu.VMEM((64, _W), jnp.float32),
            pltpu.VMEM_SHARED((_M, _W), jnp.float32),
            pltpu.SemaphoreType.DMA,
        ],
    )
    sentinel = pl.kernel(
        _sentinel_body,
        out_type=jax.ShapeDtypeStruct((_ZTOT,), jnp.float32),
        mesh=mesh,
        compiler_params=params,
        scratch_types=[
            pltpu.VMEM((8192,), jnp.float32),
            pltpu.VMEM((128,), jnp.int32),
            pltpu.VMEM((128,), jnp.int32),
            pltpu.VMEM((128,), jnp.float32),
            pltpu.VMEM((16,), jnp.int32),
            pltpu.VMEM((16,), jnp.int32),
            pltpu.VMEM((16,), jnp.float32),
        ],
    )
    return segsum, sentinel


def _scores_body(x_ref, z_ref, exp_ref, w_ref, s_ref, bscr):
    i = pl.program_id(0)

    @pl.when(i == 0)
    def _():
        ex = exp_ref[0] + exp_ref[1]
        cnt = ex[:, 128:129]
        exd = ex[:, :128] / jnp.maximum(cnt, 1.0)
        for ch in range(_NC):
            y = exd * w_ref[ch, :][None, :]
            nrm = jnp.sqrt(jnp.sum(y * y, axis=1, keepdims=True))
            bscr[:, ch * 128:(ch + 1) * 128] = y / jnp.maximum(nrm, 1e-12)

    x = x_ref[...]
    cols = []
    for ch in range(_NC):
        y = x * w_ref[ch, :][None, :]
        nrm = jnp.sqrt(jnp.sum(y * y, axis=1, keepdims=True))
        cols.append(y / jnp.maximum(nrm, 1e-12))
    a = jnp.concatenate(cols, axis=1)
    s = lax.dot_general(a, bscr[...], (((1,), (1,)), ((), ())),
                        preferred_element_type=jnp.float32)
    s_ref[...] = s * 0.25 + z_ref[...]


def _tc_scores(x, z, exp_, w):
    return pl.pallas_call(
        _scores_body,
        grid=(10,),
        in_specs=[
            pl.BlockSpec((1000, 128), lambda i: (i, 0)),
            pl.BlockSpec((1000, 1024), lambda i: (i, 0)),
            pl.BlockSpec((2, _M, _W), lambda i: (0, 0, 0)),
            pl.BlockSpec((_NC, 128), lambda i: (0, 0)),
        ],
        out_specs=pl.BlockSpec((1000, 1024), lambda i: (i, 0)),
        out_shape=jax.ShapeDtypeStruct((_N, _M), jnp.float32),
        scratch_shapes=[pltpu.VMEM((_M, _NC * 128), jnp.float32)],
    )(x, z, exp_, w)


def _count_body(edges_ref, s_ref, cnt_ref):
    i = pl.program_id(0)

    @pl.when(i == 0)
    def _():
        for j in range(16):
            cnt_ref[0, j] = 0.0

    s = s_ref[...]
    for j in range(16):
        cnt_ref[0, j] += jnp.sum((s >= edges_ref[j]).astype(jnp.float32))


def _tc_count(s, edges):
    return pl.pallas_call(
        _count_body,
        grid=(5,),
        in_specs=[
            pl.BlockSpec(memory_space=pltpu.SMEM),
            pl.BlockSpec((2000, 1024), lambda i: (i, 0)),
        ],
        out_specs=pl.BlockSpec(memory_space=pltpu.SMEM),
        out_shape=jax.ShapeDtypeStruct((1, 16), jnp.float32),
    )(edges, s)


def _final_body(t_ref, s_ref, h_ref, u_ref, p_ref, o_ref):
    t = t_ref[0]
    p = jnp.clip(p_ref[...], 1e-6, 1.0 - 1e-6)
    u = u_ref[...]
    arg = (jnp.log(p) - jnp.log1p(-p)) + (jnp.log(u) - jnp.log1p(-u))
    msk = (arg > 0.0).astype(jnp.float32)
    delta = (s_ref[...] >= t).astype(jnp.float32)
    o_ref[...] = (h_ref[...] + delta) * msk


def _tc_final(t, s, h, u, p):
    blk = pl.BlockSpec((1000, 1024), lambda i: (i, 0))
    return pl.pallas_call(
        _final_body,
        grid=(10,),
        in_specs=[pl.BlockSpec(memory_space=pltpu.SMEM), blk, blk, blk, blk],
        out_specs=blk,
        out_shape=jax.ShapeDtypeStruct((_N, _M), jnp.float32),
    )(t, s, h, u, p)


def kernel(X, H, V, E, incident_mask_prob, cos_weight):
    f32 = jnp.float32
    v32 = V.astype(jnp.int32)
    e32 = E.astype(jnp.int32)
    idxf = v32 * _M + e32
    xa = jnp.concatenate(
        [X.astype(f32),
         jnp.ones((_N, 1), f32),
         jnp.zeros((_N, _W - _EMB - 1), f32)], axis=1)

    segsum, sentinel = _sc_kernels()
    exp_ = segsum(xa, v32, e32)
    z = sentinel(idxf).reshape(_ZR, _M)
    s = _tc_scores(X, z, exp_, cos_weight)

    lo = jnp.asarray(-1.01, f32)
    hi = jnp.asarray(1.01, f32)
    kk = jnp.asarray(float(_K), f32)
    steps = jnp.arange(1, 17, dtype=f32) / 16.0
    for _ in range(5):
        edges = hi - steps * (hi - lo)
        cnts = _tc_count(s, edges)[0]
        ok = cnts >= kk
        jsel = jnp.argmax(ok)
        lo2 = edges[jsel]
        hi2 = jnp.where(jsel == 0, hi, edges[jnp.maximum(jsel - 1, 0)])
        lo, hi = lo2, hi2

    t = lo.reshape(1)
    u = jax.random.uniform(jax.random.key(42), (_N, _M),
                           minval=1e-6, maxval=1.0 - 1e-6)
    return _tc_final(t, s, H, u, incident_mask_prob)


# speedup vs baseline: 1.0004x; 1.0004x over previous
import functools

import jax
import jax.numpy as jnp
from jax import lax
from jax.experimental import pallas as pl
from jax.experimental.pallas import tpu as pltpu
from jax.experimental.pallas import tpu_sc as plsc

_N = 10000
_M = 1024
_NNZ = 160000
_EMB = 128
_NC = 4
_K = 8000
_W = 128
_ZR = _N + 16
_ZTOT = _ZR * _M
_NTILES = 32
_ZCHUNK = _ZTOT // _NTILES
_PPT_SEG = _NNZ // _NTILES
_PPT_SEN = _NNZ // 16
_SENT = jnp.float32(-2.0e30)

def _segsum_body(xa, v, e, out, idx_v, idx_e, rows, idx_v8, idx_e8, rows8,
                 zbuf, acc, sem):
    c = lax.axis_index("c")
    s = lax.axis_index("s")
    wid = c * 16 + s

    def _zero(i, _):
        r = i // 8
        j = i % 8
        zbuf[r, pl.ds(j * 16, 16)] = jnp.zeros((16,), jnp.float32)
        return 0

    lax.fori_loop(0, 64 * 8, _zero, 0)
    pltpu.sync_copy(zbuf, acc.at[pl.ds(s * 64, 64)])
    plsc.subcore_barrier()

    base0 = wid * _PPT_SEG

    def _chunk(j, _):
        b = base0 + j * 128
        pltpu.sync_copy(v.at[pl.ds(b, 128)], idx_v)
        pltpu.sync_copy(e.at[pl.ds(b, 128)], idx_e)
        pltpu.async_copy(xa.at[idx_v], rows, sem).wait()
        pltpu.sync_copy(rows, acc.at[idx_e], add=True)
        return 0

    lax.fori_loop(0, 39, _chunk, 0)
    bt = base0 + 39 * 128
    pltpu.sync_copy(v.at[pl.ds(bt, 8)], idx_v8)
    pltpu.sync_copy(e.at[pl.ds(bt, 8)], idx_e8)
    pltpu.async_copy(xa.at[idx_v8], rows8, sem).wait()
    pltpu.sync_copy(rows8, acc.at[idx_e8], add=True)

    plsc.subcore_barrier()
    pltpu.sync_copy(acc.at[pl.ds(s * 64, 64)], out.at[c, pl.ds(s * 64, 64)])


def _sentinel_body(idxf, zout, cntout, zbuf, ibuf, sbuf, ebuf, obuf, cbuf,
                   ibuf16, sbuf16, ebuf16, obuf16, cbuf16, cacc):
    c = lax.axis_index("c")
    s = lax.axis_index("s")
    wid = c * 16 + s

    def _zero(i, _):
        zbuf[pl.ds(i * 16, 16)] = jnp.zeros((16,), jnp.float32)
        return 0

    lax.fori_loop(0, 512, _zero, 0)
    for j in range(8):
        cbuf[pl.ds(j * 16, 16)] = jnp.full((16,), _SENT, jnp.float32)
        obuf[pl.ds(j * 16, 16)] = jnp.full((16,), 1.0, jnp.float32)
    cbuf16[pl.ds(0, 16)] = jnp.full((16,), _SENT, jnp.float32)
    obuf16[pl.ds(0, 16)] = jnp.full((16,), 1.0, jnp.float32)
    pltpu.sync_copy(zbuf.at[pl.ds(0, 64)], cacc.at[pl.ds(s * 64, 64)])

    zlo = wid * _ZCHUNK

    def _fill(i, _):
        pltpu.sync_copy(zbuf, zout.at[pl.ds(zlo + i * 8192, 8192)])
        return 0

    lax.fori_loop(0, 39, _fill, 0)
    pltpu.sync_copy(zbuf.at[pl.ds(0, 1024)],
                    zout.at[pl.ds(zlo + 39 * 8192, 1024)])
    plsc.subcore_barrier()

    half = _ZTOT // 2
    hlo = c * half
    hhi = hlo + half
    dump = _N * _M + wid
    pbase = s * _PPT_SEN

    def _sel(buf_i, buf_s, buf_e, nsub):
        for jj in range(nsub):
            vv = buf_i[pl.ds(jj * 16, 16)]
            inr = (vv >= hlo) & (vv < hhi)
            buf_s[pl.ds(jj * 16, 16)] = jnp.where(
                inr, vv, jnp.zeros((16,), jnp.int32) + dump)
            buf_e[pl.ds(jj * 16, 16)] = vv & (_M - 1)

    def _scat(j, _):
        b = pbase + j * 128
        pltpu.sync_copy(idxf.at[pl.ds(b, 128)], ibuf)
        _sel(ibuf, sbuf, ebuf, 8)
        pltpu.sync_copy(cbuf, zout.at[sbuf])

        @pl.when(c == 0)
        def _():
            pltpu.sync_copy(obuf, cacc.at[ebuf], add=True)

        return 0

    lax.fori_loop(0, 78, _scat, 0)
    bt = pbase + 78 * 128
    pltpu.sync_copy(idxf.at[pl.ds(bt, 16)], ibuf16)
    _sel(ibuf16, sbuf16, ebuf16, 1)
    pltpu.sync_copy(cbuf16, zout.at[sbuf16])

    @pl.when(c == 0)
    def _():
        pltpu.sync_copy(obuf16, cacc.at[ebuf16], add=True)

    plsc.subcore_barrier()

    @pl.when(c == 0)
    def _():
        pltpu.sync_copy(cacc.at[pl.ds(s * 64, 64)],
                        cntout.at[pl.ds(s * 64, 64)])


@functools.cache
def _sc_kernels():
    mesh = plsc.VectorSubcoreMesh(core_axis_name="c", subcore_axis_name="s",
                                  num_cores=2, num_subcores=16)
    segsum = pl.kernel(
        _segsum_body,
        out_type=jax.ShapeDtypeStruct((2, _M, _W), jnp.float32),
        mesh=mesh,
        scratch_types=[
            pltpu.VMEM((128,), jnp.int32),
            pltpu.VMEM((128,), jnp.int32),
            pltpu.VMEM((128, _W), jnp.float32),
            pltpu.VMEM((8,), jnp.int32),
            pltpu.VMEM((8,), jnp.int32),
            pltpu.VMEM((8, _W), jnp.float32),
            pltpu.VMEM((64, _W), jnp.float32),
            pltpu.VMEM_SHARED((_M, _W), jnp.float32),
            pltpu.SemaphoreType.DMA,
        ],
    )
    sentinel = pl.kernel(
        _sentinel_body,
        out_type=(jax.ShapeDtypeStruct((_ZTOT,), jnp.float32),
                  jax.ShapeDtypeStruct((_M,), jnp.float32)),
        mesh=mesh,
        compiler_params=pltpu.CompilerParams(use_tc_tiling_on_sc=False),
        scratch_types=[
            pltpu.VMEM((8192,), jnp.float32),
            pltpu.VMEM((128,), jnp.int32),
            pltpu.VMEM((128,), jnp.int32),
            pltpu.VMEM((128,), jnp.int32),
            pltpu.VMEM((128,), jnp.float32),
            pltpu.VMEM((128,), jnp.float32),
            pltpu.VMEM((16,), jnp.int32),
            pltpu.VMEM((16,), jnp.int32),
            pltpu.VMEM((16,), jnp.int32),
            pltpu.VMEM((16,), jnp.float32),
            pltpu.VMEM((16,), jnp.float32),
            pltpu.VMEM_SHARED((_M,), jnp.float32),
        ],
    )
    return segsum, sentinel


def _scores_body(x_ref, z_ref, exp_ref, cnt_ref, w_ref, s_ref, bscr):
    i = pl.program_id(0)

    @pl.when(i == 0)
    def _():
        ex = exp_ref[0] + exp_ref[1]
        exd = ex / jnp.maximum(cnt_ref[...], 1.0)
        for ch in range(_NC):
            y = exd * w_ref[ch, :][None, :]
            nrm = jnp.sqrt(jnp.sum(y * y, axis=1, keepdims=True))
            bscr[:, ch * 128:(ch + 1) * 128] = y / jnp.maximum(nrm, 1e-12)

    x = x_ref[...]
    cols = []
    for ch in range(_NC):
        y = x * w_ref[ch, :][None, :]
        nrm = jnp.sqrt(jnp.sum(y * y, axis=1, keepdims=True))
        cols.append(y / jnp.maximum(nrm, 1e-12))
    a = jnp.concatenate(cols, axis=1)
    s = lax.dot_general(a, bscr[...], (((1,), (1,)), ((), ())),
                        preferred_element_type=jnp.float32)
    s_ref[...] = s * 0.25 + z_ref[...]


def _tc_scores(x, z, exp_, cnt, w):
    return pl.pallas_call(
        _scores_body,
        grid=(10,),
        in_specs=[
            pl.BlockSpec((1000, 128), lambda i: (i, 0)),
            pl.BlockSpec((1000, 1024), lambda i: (i, 0)),
            pl.BlockSpec((2, _M, _W), lambda i: (0, 0, 0)),
            pl.BlockSpec((_M, 1), lambda i: (0, 0)),
            pl.BlockSpec((_NC, 128), lambda i: (0, 0)),
        ],
        out_specs=pl.BlockSpec((1000, 1024), lambda i: (i, 0)),
        out_shape=jax.ShapeDtypeStruct((_N, _M), jnp.float32),
        scratch_shapes=[pltpu.VMEM((_M, _NC * 128), jnp.float32)],
    )(x, z, exp_, cnt, w)


def _count_body(edges_ref, s_ref, cnt_ref):
    i = pl.program_id(0)

    @pl.when(i == 0)
    def _():
        for j in range(16):
            cnt_ref[0, j] = 0.0

    s = s_ref[...]
    for j in range(16):
        cnt_ref[0, j] += jnp.sum((s >= edges_ref[j]).astype(jnp.float32))


def _tc_count(s, edges):
    return pl.pallas_call(
        _count_body,
        grid=(5,),
        in_specs=[
            pl.BlockSpec(memory_space=pltpu.SMEM),
            pl.BlockSpec((2000, 1024), lambda i: (i, 0)),
        ],
        out_specs=pl.BlockSpec(memory_space=pltpu.SMEM),
        out_shape=jax.ShapeDtypeStruct((1, 16), jnp.float32),
    )(edges, s)


def _final_body(t_ref, s_ref, h_ref, u_ref, p_ref, o_ref):
    t = t_ref[0]
    p = jnp.clip(p_ref[...], 1e-6, 1.0 - 1e-6)
    u = u_ref[...]
    arg = (jnp.log(p) - jnp.log1p(-p)) + (jnp.log(u) - jnp.log1p(-u))
    msk = (arg > 0.0).astype(jnp.float32)
    delta = (s_ref[...] >= t).astype(jnp.float32)
    o_ref[...] = (h_ref[...] + delta) * msk


def _tc_final(t, s, h, u, p):
    blk = pl.BlockSpec((1000, 1024), lambda i: (i, 0))
    return pl.pallas_call(
        _final_body,
        grid=(10,),
        in_specs=[pl.BlockSpec(memory_space=pltpu.SMEM), blk, blk, blk, blk],
        out_specs=blk,
        out_shape=jax.ShapeDtypeStruct((_N, _M), jnp.float32),
    )(t, s, h, u, p)


def kernel(X, H, V, E, incident_mask_prob, cos_weight):
    f32 = jnp.float32
    v32 = V.astype(jnp.int32)
    e32 = E.astype(jnp.int32)
    idxf = v32 * _M + e32

    segsum, sentinel = _sc_kernels()
    exp_ = segsum(X, v32, e32)
    zflat, cnt = sentinel(idxf)
    z = zflat.reshape(_ZR, _M)
    s = _tc_scores(X, z, exp_, cnt.reshape(_M, 1), cos_weight)

    lo = jnp.asarray(-1.01, f32)
    hi = jnp.asarray(1.01, f32)
    kk = jnp.asarray(float(_K), f32)
    steps = jnp.arange(1, 17, dtype=f32) / 16.0
    for _ in range(5):
        edges = hi - steps * (hi - lo)
        cnts = _tc_count(s, edges)[0]
        ok = cnts >= kk
        jsel = jnp.argmax(ok)
        lo2 = edges[jsel]
        hi2 = jnp.where(jsel == 0, hi, edges[jnp.maximum(jsel - 1, 0)])
        lo, hi = lo2, hi2

    t = lo.reshape(1)
    u = jax.random.uniform(jax.random.key(42), (_N, _M),
                           minval=1e-6, maxval=1.0 - 1e-6)
    return _tc_final(t, s, H, u, incident_mask_prob)


# speedup vs baseline: 17.9923x; 17.9843x over previous
import functools

import jax
import jax.numpy as jnp
from jax import lax
from jax.experimental import pallas as pl
from jax.experimental.pallas import tpu as pltpu
from jax.experimental.pallas import tpu_sc as plsc

_N = 10000
_M = 1024
_NNZ = 160000
_EMB = 128
_NC = 4
_K = 8000
_W = 128
_ZR = _N + 16
_ZTOT = _ZR * _M
_NTILES = 32
_ZCHUNK = _ZTOT // _NTILES
_NPH = 4
_ZPH = _ZCHUNK // _NPH
_PPT_SEG = _NNZ // _NTILES
_SENT = jnp.float32(-2.0e30)

def _segsum_body(xa, v, e, out, idx_v, idx_e, rows, idx_v8, idx_e8, rows8,
                 zbuf, onesb, acc, acc2, sem):
    c = lax.axis_index("c")
    s = lax.axis_index("s")
    wid = c * 16 + s

    def _zero(i, _):
        r = i // 8
        j = i % 8
        zbuf[r, pl.ds(j * 16, 16)] = jnp.zeros((16,), jnp.float32)
        return 0

    lax.fori_loop(0, 64 * 8, _zero, 0)

    def _ones(i, _):
        r = i // 8
        j = i % 8
        onesb[r, pl.ds(j * 16, 16)] = jnp.full((16,), 1.0, jnp.float32)
        return 0

    lax.fori_loop(0, 128 * 8, _ones, 0)
    pltpu.sync_copy(zbuf, acc.at[pl.ds(s * 64, 64)])
    pltpu.sync_copy(zbuf, acc2.at[pl.ds(s * 64, 64)])
    plsc.subcore_barrier()

    base0 = wid * _PPT_SEG

    def _chunk(j, _):
        b = base0 + j * 128
        pltpu.sync_copy(v.at[pl.ds(b, 128)], idx_v)
        pltpu.sync_copy(e.at[pl.ds(b, 128)], idx_e)
        pltpu.async_copy(xa.at[idx_v], rows, sem).wait()
        pltpu.sync_copy(rows, acc.at[idx_e], add=True)
        pltpu.sync_copy(onesb, acc2.at[idx_e], add=True)
        return 0

    lax.fori_loop(0, 39, _chunk, 0)
    bt = base0 + 39 * 128
    pltpu.sync_copy(v.at[pl.ds(bt, 8)], idx_v8)
    pltpu.sync_copy(e.at[pl.ds(bt, 8)], idx_e8)
    pltpu.async_copy(xa.at[idx_v8], rows8, sem).wait()
    pltpu.sync_copy(rows8, acc.at[idx_e8], add=True)
    pltpu.sync_copy(onesb.at[pl.ds(0, 8)], acc2.at[idx_e8], add=True)

    plsc.subcore_barrier()
    pltpu.sync_copy(acc.at[pl.ds(s * 64, 64)], out.at[c, pl.ds(s * 64, 64)])
    pltpu.sync_copy(acc2.at[pl.ds(s * 64, 64)],
                    out.at[c, pl.ds(_M + s * 64, 64)])


def _sentinel_body(idxf, zout, zc, ibuf):
    c = lax.axis_index("c")
    s = lax.axis_index("s")
    wid = c * 16 + s
    base = wid * _ZCHUNK
    sent = jnp.full((16,), _SENT, jnp.float32)

    for p in range(_NPH):
        pb = base + p * _ZPH

        def _zero(i, _):
            zc[pl.ds(i * 16, 16)] = jnp.zeros((16,), jnp.float32)
            return 0

        lax.fori_loop(0, _ZPH // 16, _zero, 0)

        def _scan(jc, _):
            pltpu.sync_copy(idxf.at[pl.ds(jc * 2000, 2000)], ibuf)

            def _vec(jj, _2):
                vv = ibuf[pl.ds(jj * 16, 16)]
                loc = vv - pb
                inr = (loc >= 0) & (loc < _ZPH)
                locc = jnp.clip(loc, 0, _ZPH - 1)
                plsc.store_scatter(zc, [locc], sent, mask=inr)
                return 0

            lax.fori_loop(0, 125, _vec, 0)
            return 0

        lax.fori_loop(0, _NNZ // 2000, _scan, 0)
        pltpu.sync_copy(zc, zout.at[pl.ds(pb, _ZPH)])


@functools.cache
def _sc_kernels():
    mesh = plsc.VectorSubcoreMesh(core_axis_name="c", subcore_axis_name="s",
                                  num_cores=2, num_subcores=16)
    segsum = pl.kernel(
        _segsum_body,
        out_type=jax.ShapeDtypeStruct((2, 2 * _M, _W), jnp.float32),
        mesh=mesh,
        scratch_types=[
            pltpu.VMEM((128,), jnp.int32),
            pltpu.VMEM((128,), jnp.int32),
            pltpu.VMEM((128, _W), jnp.float32),
            pltpu.VMEM((8,), jnp.int32),
            pltpu.VMEM((8,), jnp.int32),
            pltpu.VMEM((8, _W), jnp.float32),
            pltpu.VMEM((64, _W), jnp.float32),
            pltpu.VMEM((128, _W), jnp.float32),
            pltpu.VMEM_SHARED((_M, _W), jnp.float32),
            pltpu.VMEM_SHARED((_M, _W), jnp.float32),
            pltpu.SemaphoreType.DMA,
        ],
    )
    sentinel = pl.kernel(
        _sentinel_body,
        out_type=jax.ShapeDtypeStruct((_ZTOT,), jnp.float32),
        mesh=mesh,
        compiler_params=pltpu.CompilerParams(needs_layout_passes=False),
        scratch_types=[
            pltpu.VMEM((_ZPH,), jnp.float32),
            pltpu.VMEM((2000,), jnp.int32),
        ],
    )
    return segsum, sentinel


def _scores_body(x_ref, z_ref, exp_ref, w_ref, s_ref, bscr):
    i = pl.program_id(0)

    @pl.when(i == 0)
    def _():
        ex = exp_ref[0] + exp_ref[1]
        exd = ex[:_M] / jnp.maximum(ex[_M:, 0:1], 1.0)
        for ch in range(_NC):
            y = exd * w_ref[ch, :][None, :]
            nrm = jnp.sqrt(jnp.sum(y * y, axis=1, keepdims=True))
            bscr[:, ch * 128:(ch + 1) * 128] = y / jnp.maximum(nrm, 1e-12)

    x = x_ref[...]
    cols = []
    for ch in range(_NC):
        y = x * w_ref[ch, :][None, :]
        nrm = jnp.sqrt(jnp.sum(y * y, axis=1, keepdims=True))
        cols.append(y / jnp.maximum(nrm, 1e-12))
    a = jnp.concatenate(cols, axis=1)
    s = lax.dot_general(a, bscr[...], (((1,), (1,)), ((), ())),
                        preferred_element_type=jnp.float32)
    s_ref[...] = s * 0.25 + z_ref[...]


def _tc_scores(x, z, exp_, w):
    return pl.pallas_call(
        _scores_body,
        grid=(10,),
        in_specs=[
            pl.BlockSpec((1000, 128), lambda i: (i, 0)),
            pl.BlockSpec((1000, 1024), lambda i: (i, 0)),
            pl.BlockSpec((2, 2 * _M, _W), lambda i: (0, 0, 0)),
            pl.BlockSpec((_NC, 128), lambda i: (0, 0)),
        ],
        out_specs=pl.BlockSpec((1000, 1024), lambda i: (i, 0)),
        out_shape=jax.ShapeDtypeStruct((_N, _M), jnp.float32),
        scratch_shapes=[pltpu.VMEM((_M, _NC * 128), jnp.float32)],
    )(x, z, exp_, w)


def _count_body(edges_ref, s_ref, cnt_ref):
    i = pl.program_id(0)

    @pl.when(i == 0)
    def _():
        for j in range(16):
            cnt_ref[0, j] = 0.0

    s = s_ref[...]
    for j in range(16):
        cnt_ref[0, j] += jnp.sum((s >= edges_ref[j]).astype(jnp.float32))


def _tc_count(s, edges):
    return pl.pallas_call(
        _count_body,
        grid=(5,),
        in_specs=[
            pl.BlockSpec(memory_space=pltpu.SMEM),
            pl.BlockSpec((2000, 1024), lambda i: (i, 0)),
        ],
        out_specs=pl.BlockSpec(memory_space=pltpu.SMEM),
        out_shape=jax.ShapeDtypeStruct((1, 16), jnp.float32),
    )(edges, s)


def _final_body(t_ref, s_ref, h_ref, u_ref, p_ref, o_ref):
    t = t_ref[0]
    p = jnp.clip(p_ref[...], 1e-6, 1.0 - 1e-6)
    u = u_ref[...]
    arg = (jnp.log(p) - jnp.log1p(-p)) + (jnp.log(u) - jnp.log1p(-u))
    msk = (arg > 0.0).astype(jnp.float32)
    delta = (s_ref[...] >= t).astype(jnp.float32)
    o_ref[...] = (h_ref[...] + delta) * msk


def _tc_final(t, s, h, u, p):
    blk = pl.BlockSpec((1000, 1024), lambda i: (i, 0))
    return pl.pallas_call(
        _final_body,
        grid=(10,),
        in_specs=[pl.BlockSpec(memory_space=pltpu.SMEM), blk, blk, blk, blk],
        out_specs=blk,
        out_shape=jax.ShapeDtypeStruct((_N, _M), jnp.float32),
    )(t, s, h, u, p)


def kernel(X, H, V, E, incident_mask_prob, cos_weight):
    f32 = jnp.float32
    v32 = V.astype(jnp.int32)
    e32 = E.astype(jnp.int32)
    idxf = v32 * _M + e32

    segsum, sentinel = _sc_kernels()
    exp_ = segsum(X, v32, e32)
    z = sentinel(idxf).reshape(_ZR, _M)
    s = _tc_scores(X, z, exp_, cos_weight)

    lo = jnp.asarray(-1.01, f32)
    hi = jnp.asarray(1.01, f32)
    kk = jnp.asarray(float(_K), f32)
    steps = jnp.arange(1, 17, dtype=f32) / 16.0
    for _ in range(5):
        edges = hi - steps * (hi - lo)
        cnts = _tc_count(s, edges)[0]
        ok = cnts >= kk
        jsel = jnp.argmax(ok)
        lo2 = edges[jsel]
        hi2 = jnp.where(jsel == 0, hi, edges[jnp.maximum(jsel - 1, 0)])
        lo, hi = lo2, hi2

    t = lo.reshape(1)
    u = jax.random.uniform(jax.random.key(42), (_N, _M),
                           minval=1e-6, maxval=1.0 - 1e-6)
    return _tc_final(t, s, H, u, incident_mask_prob)


# speedup vs baseline: 23.4524x; 1.3035x over previous
import functools

import jax
import jax.numpy as jnp
from jax import lax
from jax.experimental import pallas as pl
from jax.experimental.pallas import tpu as pltpu
from jax.experimental.pallas import tpu_sc as plsc

_N = 10000
_M = 1024
_NNZ = 160000
_EMB = 128
_NC = 4
_K = 8000
_W = 128
_ZR = _N + 16
_ZTOT = _ZR * _M
_NTILES = 32
_ZCHUNK = _ZTOT // _NTILES
_NPH = 4
_ZPH = _ZCHUNK // _NPH
_PPT_SEG = _NNZ // _NTILES
_SENT = jnp.float32(-2.0e30)

def _segsum_body(xa, v, e, out, idx_v, idx_e, rows, idx_v8, idx_e8, rows8,
                 zbuf, onesb, acc, acc2, sem):
    c = lax.axis_index("c")
    s = lax.axis_index("s")
    wid = c * 16 + s

    def _zero(i, _):
        r = i // 8
        j = i % 8
        zbuf[r, pl.ds(j * 16, 16)] = jnp.zeros((16,), jnp.float32)
        return 0

    lax.fori_loop(0, 64 * 8, _zero, 0)

    def _ones(i, _):
        r = i // 8
        j = i % 8
        onesb[r, pl.ds(j * 16, 16)] = jnp.full((16,), 1.0, jnp.float32)
        return 0

    lax.fori_loop(0, 128 * 8, _ones, 0)
    pltpu.sync_copy(zbuf, acc.at[pl.ds(s * 64, 64)])
    pltpu.sync_copy(zbuf, acc2.at[pl.ds(s * 64, 64)])
    plsc.subcore_barrier()

    base0 = wid * _PPT_SEG

    def _chunk(j, _):
        b = base0 + j * 128
        pltpu.sync_copy(v.at[pl.ds(b, 128)], idx_v)
        pltpu.sync_copy(e.at[pl.ds(b, 128)], idx_e)
        pltpu.async_copy(xa.at[idx_v], rows, sem).wait()
        pltpu.sync_copy(rows, acc.at[idx_e], add=True)
        pltpu.sync_copy(onesb, acc2.at[idx_e], add=True)
        return 0

    lax.fori_loop(0, 39, _chunk, 0)
    bt = base0 + 39 * 128
    pltpu.sync_copy(v.at[pl.ds(bt, 8)], idx_v8)
    pltpu.sync_copy(e.at[pl.ds(bt, 8)], idx_e8)
    pltpu.async_copy(xa.at[idx_v8], rows8, sem).wait()
    pltpu.sync_copy(rows8, acc.at[idx_e8], add=True)
    pltpu.sync_copy(onesb.at[pl.ds(0, 8)], acc2.at[idx_e8], add=True)

    plsc.subcore_barrier()
    pltpu.sync_copy(acc.at[pl.ds(s * 64, 64)], out.at[c, pl.ds(s * 64, 64)])
    pltpu.sync_copy(acc2.at[pl.ds(s * 64, 64)],
                    out.at[c, pl.ds(_M + s * 64, 64)])


def _sentinel_body(idxf, zout, zc, ibuf, ibuf2, sem):
    c = lax.axis_index("c")
    s = lax.axis_index("s")
    wid = c * 16 + s
    base = wid * _ZCHUNK
    sent = jnp.full((16,), _SENT, jnp.float32)
    nch = _NNZ // 2000

    for p in range(_NPH):
        pb = base + p * _ZPH

        def _zero(i, _):
            for q in range(8):
                zc[pl.ds(i * 128 + q * 16, 16)] = jnp.zeros((16,),
                                                            jnp.float32)
            return 0

        lax.fori_loop(0, _ZPH // 128, _zero, 0)

        def _scansub(buf):
            for jj in range(125):
                vv = buf[pl.ds(jj * 16, 16)]
                loc = vv - pb
                inr = (loc >= 0) & (loc < _ZPH)
                locc = jnp.clip(loc, 0, _ZPH - 1)
                plsc.store_scatter(zc, [locc], sent, mask=inr)

        pltpu.async_copy(idxf.at[pl.ds(0, 2000)], ibuf, sem)

        def _scan(j, _):
            jc = j * 2
            pltpu.make_async_copy(idxf.at[pl.ds(jc * 2000, 2000)],
                                  ibuf, sem).wait()
            pltpu.async_copy(idxf.at[pl.ds((jc + 1) * 2000, 2000)],
                             ibuf2, sem)
            _scansub(ibuf)
            pltpu.make_async_copy(idxf.at[pl.ds((jc + 1) * 2000, 2000)],
                                  ibuf2, sem).wait()

            @pl.when(j < nch // 2 - 1)
            def _():
                pltpu.async_copy(idxf.at[pl.ds((jc + 2) * 2000, 2000)],
                                 ibuf, sem)

            _scansub(ibuf2)
            return 0

        lax.fori_loop(0, nch // 2, _scan, 0)
        pltpu.sync_copy(zc, zout.at[pl.ds(pb, _ZPH)])


@functools.cache
def _sc_kernels():
    mesh = plsc.VectorSubcoreMesh(core_axis_name="c", subcore_axis_name="s",
                                  num_cores=2, num_subcores=16)
    segsum = pl.kernel(
        _segsum_body,
        out_type=jax.ShapeDtypeStruct((2, 2 * _M, _W), jnp.float32),
        mesh=mesh,
        scratch_types=[
            pltpu.VMEM((128,), jnp.int32),
            pltpu.VMEM((128,), jnp.int32),
            pltpu.VMEM((128, _W), jnp.float32),
            pltpu.VMEM((8,), jnp.int32),
            pltpu.VMEM((8,), jnp.int32),
            pltpu.VMEM((8, _W), jnp.float32),
            pltpu.VMEM((64, _W), jnp.float32),
            pltpu.VMEM((128, _W), jnp.float32),
            pltpu.VMEM_SHARED((_M, _W), jnp.float32),
            pltpu.VMEM_SHARED((_M, _W), jnp.float32),
            pltpu.SemaphoreType.DMA,
        ],
    )
    sentinel = pl.kernel(
        _sentinel_body,
        out_type=jax.ShapeDtypeStruct((_ZTOT,), jnp.float32),
        mesh=mesh,
        compiler_params=pltpu.CompilerParams(needs_layout_passes=False),
        scratch_types=[
            pltpu.VMEM((_ZPH,), jnp.float32),
            pltpu.VMEM((2000,), jnp.int32),
            pltpu.VMEM((2000,), jnp.int32),
            pltpu.SemaphoreType.DMA,
        ],
    )
    return segsum, sentinel


def _scores_body(x_ref, z_ref, exp_ref, w_ref, s_ref, bscr):
    i = pl.program_id(0)

    @pl.when(i == 0)
    def _():
        ex = exp_ref[0] + exp_ref[1]
        exd = ex[:_M] / jnp.maximum(ex[_M:, 0:1], 1.0)
        for ch in range(_NC):
            y = exd * w_ref[ch, :][None, :]
            nrm = jnp.sqrt(jnp.sum(y * y, axis=1, keepdims=True))
            bscr[:, ch * 128:(ch + 1) * 128] = y / jnp.maximum(nrm, 1e-12)

    x = x_ref[...]
    cols = []
    for ch in range(_NC):
        y = x * w_ref[ch, :][None, :]
        nrm = jnp.sqrt(jnp.sum(y * y, axis=1, keepdims=True))
        cols.append(y / jnp.maximum(nrm, 1e-12))
    a = jnp.concatenate(cols, axis=1)
    s = lax.dot_general(a, bscr[...], (((1,), (1,)), ((), ())),
                        preferred_element_type=jnp.float32)
    s_ref[...] = s * 0.25 + z_ref[...]


def _tc_scores(x, z, exp_, w):
    return pl.pallas_call(
        _scores_body,
        grid=(10,),
        in_specs=[
            pl.BlockSpec((1000, 128), lambda i: (i, 0)),
            pl.BlockSpec((1000, 1024), lambda i: (i, 0)),
            pl.BlockSpec((2, 2 * _M, _W), lambda i: (0, 0, 0)),
            pl.BlockSpec((_NC, 128), lambda i: (0, 0)),
        ],
        out_specs=pl.BlockSpec((1000, 1024), lambda i: (i, 0)),
        out_shape=jax.ShapeDtypeStruct((_N, _M), jnp.float32),
        scratch_shapes=[pltpu.VMEM((_M, _NC * 128), jnp.float32)],
    )(x, z, exp_, w)


def _count_body(edges_ref, s_ref, cnt_ref):
    i = pl.program_id(0)

    @pl.when(i == 0)
    def _():
        for j in range(16):
            cnt_ref[0, j] = 0.0

    s = s_ref[...]
    for j in range(16):
        cnt_ref[0, j] += jnp.sum((s >= edges_ref[j]).astype(jnp.float32))


def _tc_count(s, edges):
    return pl.pallas_call(
        _count_body,
        grid=(5,),
        in_specs=[
            pl.BlockSpec(memory_space=pltpu.SMEM),
            pl.BlockSpec((2000, 1024), lambda i: (i, 0)),
        ],
        out_specs=pl.BlockSpec(memory_space=pltpu.SMEM),
        out_shape=jax.ShapeDtypeStruct((1, 16), jnp.float32),
    )(edges, s)


def _final_body(t_ref, s_ref, h_ref, u_ref, p_ref, o_ref):
    t = t_ref[0]
    p = jnp.clip(p_ref[...], 1e-6, 1.0 - 1e-6)
    u = u_ref[...]
    arg = (jnp.log(p) - jnp.log1p(-p)) + (jnp.log(u) - jnp.log1p(-u))
    msk = (arg > 0.0).astype(jnp.float32)
    delta = (s_ref[...] >= t).astype(jnp.float32)
    o_ref[...] = (h_ref[...] + delta) * msk


def _tc_final(t, s, h, u, p):
    blk = pl.BlockSpec((1000, 1024), lambda i: (i, 0))
    return pl.pallas_call(
        _final_body,
        grid=(10,),
        in_specs=[pl.BlockSpec(memory_space=pltpu.SMEM), blk, blk, blk, blk],
        out_specs=blk,
        out_shape=jax.ShapeDtypeStruct((_N, _M), jnp.float32),
    )(t, s, h, u, p)


def kernel(X, H, V, E, incident_mask_prob, cos_weight):
    f32 = jnp.float32
    v32 = V.astype(jnp.int32)
    e32 = E.astype(jnp.int32)
    idxf = v32 * _M + e32

    segsum, sentinel = _sc_kernels()
    exp_ = segsum(X, v32, e32)
    z = sentinel(idxf).reshape(_ZR, _M)
    s = _tc_scores(X, z, exp_, cos_weight)

    lo = jnp.asarray(-1.01, f32)
    hi = jnp.asarray(1.01, f32)
    kk = jnp.asarray(float(_K), f32)
    steps = jnp.arange(1, 17, dtype=f32) / 16.0
    for _ in range(4):
        edges = hi - steps * (hi - lo)
        cnts = _tc_count(s, edges)[0]
        ok = cnts >= kk
        jsel = jnp.argmax(ok)
        lo2 = edges[jsel]
        hi2 = jnp.where(jsel == 0, hi, edges[jnp.maximum(jsel - 1, 0)])
        lo, hi = lo2, hi2

    t = lo.reshape(1)
    u = jax.random.uniform(jax.random.key(42), (_N, _M),
                           minval=1e-6, maxval=1.0 - 1e-6)
    return _tc_final(t, s, H, u, incident_mask_prob)
